# baseline (speedup 1.0000x reference)
import functools

import jax
import jax.numpy as jnp
from jax import lax
from jax.experimental import pallas as pl
from jax.experimental.pallas import tpu as pltpu
from jax.experimental.pallas import tpu_sc as plsc

_M = 100000
_C = 100
_D = 128
_B = 16384

_info = plsc.get_sparse_core_info()
_NC, _NS = _info.num_cores, _info.num_subcores
_NW = _NC * _NS
_BPW = _B // _NW
_CHUNK = 128
_NCH = _BPW // _CHUNK


@functools.partial(
    pl.kernel,
    mesh=plsc.VectorSubcoreMesh(core_axis_name="c", subcore_axis_name="s"),
    compiler_params=pltpu.CompilerParams(use_tc_tiling_on_sc=False),
    out_type=[
        jax.ShapeDtypeStruct((_B, _C), jnp.float32),
        jax.ShapeDtypeStruct((_B, _C), jnp.float32),
        jax.ShapeDtypeStruct((_B, _D), jnp.float32),
        jax.ShapeDtypeStruct((_B, _D), jnp.float32),
    ],
    scratch_types=[
        pltpu.VMEM((_NCH, _CHUNK), jnp.int32),
        pltpu.VMEM((_BPW, _C), jnp.float32),
        pltpu.VMEM((_BPW, _D), jnp.float32),
        pltpu.SemaphoreType.DMA,
        pltpu.SemaphoreType.DMA,
        pltpu.SemaphoreType.DMA,
        pltpu.SemaphoreType.DMA,
    ],
)
def _gather4(index_hbm, wl_hbm, wf_hbm, sl_hbm, sf_hbm,
             out_wl, out_sl, out_wf, out_sf,
             idx_v, lbuf, fbuf, sem_gl, sem_gf, sem_wl, sem_wf):
    wid = lax.axis_index("s") * _NC + lax.axis_index("c")
    base = wid * _BPW
    pltpu.sync_copy(index_hbm.at[pl.ds(wid * _NCH, _NCH)], idx_v)

    def gather(tbl, buf, sem):
        cps = []
        for j in range(_NCH):
            cps.append(pltpu.async_copy(
                tbl.at[idx_v.at[j]], buf.at[pl.ds(j * _CHUNK, _CHUNK)], sem))
        return cps

    def drain(cps):
        for cp in cps:
            cp.wait()

    g_wl = gather(wl_hbm, lbuf, sem_gl)
    g_wf = gather(wf_hbm, fbuf, sem_gf)
    drain(g_wl)
    wb_wl = pltpu.async_copy(lbuf, out_wl.at[pl.ds(base, _BPW)], sem_wl)
    drain(g_wf)
    wb_wf = pltpu.async_copy(fbuf, out_wf.at[pl.ds(base, _BPW)], sem_wf)
    wb_wl.wait()
    g_sl = gather(sl_hbm, lbuf, sem_gl)
    wb_wf.wait()
    g_sf = gather(sf_hbm, fbuf, sem_gf)
    drain(g_sl)
    wb_sl = pltpu.async_copy(lbuf, out_sl.at[pl.ds(base, _BPW)], sem_wl)
    drain(g_sf)
    wb_sf = pltpu.async_copy(fbuf, out_sf.at[pl.ds(base, _BPW)], sem_wf)
    wb_sl.wait()
    wb_sf.wait()


def kernel(x, index, weak_logits_mem, weak_features_mem,
           strong_logits_mem, strong_features_mem):
    del x
    index2d = index.reshape(_B // _CHUNK, _CHUNK)
    wl, sl, wf, sf = _gather4(index2d, weak_logits_mem, weak_features_mem,
                              strong_logits_mem, strong_features_mem)
    return ([wl, sl], [wf, sf])


# speedup vs baseline: 1.1677x; 1.1677x over previous
import functools

import jax
import jax.numpy as jnp
from jax import lax
from jax.experimental import pallas as pl
from jax.experimental.pallas import tpu as pltpu
from jax.experimental.pallas import tpu_sc as plsc

_M = 100000
_C = 100
_D = 128
_B = 16384

_info = plsc.get_sparse_core_info()
_NC, _NS = _info.num_cores, _info.num_subcores
_NW = _NC * _NS
_BPW = _B // _NW
_CHUNK = 128
_NCH = _BPW // _CHUNK
_HALF = _BPW // 2
_NBUF = 3


@functools.partial(
    pl.kernel,
    mesh=plsc.VectorSubcoreMesh(core_axis_name="c", subcore_axis_name="s"),
    out_type=[
        jax.ShapeDtypeStruct((_B, _D), jnp.float32),
        jax.ShapeDtypeStruct((_B, _D), jnp.float32),
        jax.ShapeDtypeStruct((_B, _D), jnp.float32),
        jax.ShapeDtypeStruct((_B, _D), jnp.float32),
    ],
    scratch_types=[
        pltpu.VMEM((_NCH, _CHUNK), jnp.int32),
        pltpu.VMEM((_NBUF, _HALF, _D), jnp.float32),
        pltpu.SemaphoreType.DMA,
        pltpu.SemaphoreType.DMA,
        pltpu.SemaphoreType.DMA,
        pltpu.SemaphoreType.DMA,
        pltpu.SemaphoreType.DMA,
        pltpu.SemaphoreType.DMA,
    ],
)
def _gather4(index_hbm, wl_hbm, wf_hbm, sl_hbm, sf_hbm,
             out_wl, out_sl, out_wf, out_sf,
             idx_v, bufs, *sems):
    sem_g = sems[:_NBUF]
    sem_w = sems[_NBUF:]
    wid = lax.axis_index("s") * _NC + lax.axis_index("c")
    base = wid * _BPW
    pltpu.sync_copy(index_hbm.at[wid], idx_v)
    my_idx = idx_v

    tasks = [(wl_hbm, out_wl, 0), (wf_hbm, out_wf, 0),
             (sl_hbm, out_sl, 0), (sf_hbm, out_sf, 0),
             (wl_hbm, out_wl, 1), (wf_hbm, out_wf, 1),
             (sl_hbm, out_sl, 1), (sf_hbm, out_sf, 1)]

    gathers = [None] * len(tasks)
    writes = [None] * len(tasks)

    def fire_gather(t):
        tbl, _, h = tasks[t]
        b = t % _NBUF
        cps = []
        for j in (2 * h, 2 * h + 1):
            dst_row = (j - 2 * h) * _CHUNK
            cps.append(pltpu.async_copy(
                tbl.at[my_idx.at[j]],
                bufs.at[b].at[pl.ds(dst_row, _CHUNK)],
                sem_g[b]))
        gathers[t] = cps

    def fire_write(t):
        _, out, h = tasks[t]
        b = t % _NBUF
        for cp in gathers[t]:
            cp.wait()
        writes[t] = pltpu.async_copy(
            bufs.at[b], out.at[pl.ds(base + h * _HALF, _HALF)], sem_w[b])

    for t in range(len(tasks)):
        if t >= _NBUF:
            writes[t - _NBUF].wait()
        fire_gather(t)
        if t >= 1:
            fire_write(t - 1)
    fire_write(len(tasks) - 1)
    for t in range(len(tasks) - _NBUF, len(tasks)):
        writes[t].wait()


def kernel(x, index, weak_logits_mem, weak_features_mem,
           strong_logits_mem, strong_features_mem):
    del x
    index3d = index.reshape(_NW, _NCH, _CHUNK)
    pad = ((0, 0), (0, _D - _C))
    wl_p = jnp.pad(weak_logits_mem, pad)
    sl_p = jnp.pad(strong_logits_mem, pad)
    wl, sl, wf, sf = _gather4(index3d, wl_p, weak_features_mem,
                              sl_p, strong_features_mem)
    return ([wl[:, :_C], sl[:, :_C]], [wf, sf])


# speedup vs baseline: 2.2994x; 1.9691x over previous
import functools

import jax
import jax.numpy as jnp
from jax import lax
from jax.experimental import pallas as pl
from jax.experimental.pallas import tpu as pltpu
from jax.experimental.pallas import tpu_sc as plsc

_M = 100000
_C = 100
_D = 128
_B = 16384

_info = plsc.get_sparse_core_info()
_NC, _NS = _info.num_cores, _info.num_subcores
_NW = _NC * _NS
_BPW = _B // _NW
_CHUNK = 128
_NCH = _BPW // _CHUNK
_HALF = _BPW // 2
_NBUF = 3

_PAD_ROWS = 4000


def _pad_body(wl_ref, sl_ref, wlp_ref, slp_ref):
    cfg = ((0, 0, 0), (0, _D - _C, 0))
    wlp_ref[...] = lax.pad(wl_ref[...], jnp.float32(0), cfg)
    slp_ref[...] = lax.pad(sl_ref[...], jnp.float32(0), cfg)


_pad2 = pl.pallas_call(
    _pad_body,
    grid=(_M // _PAD_ROWS,),
    in_specs=[
        pl.BlockSpec((_PAD_ROWS, _C), lambda i: (i, 0)),
        pl.BlockSpec((_PAD_ROWS, _C), lambda i: (i, 0)),
    ],
    out_specs=[
        pl.BlockSpec((_PAD_ROWS, _D), lambda i: (i, 0)),
        pl.BlockSpec((_PAD_ROWS, _D), lambda i: (i, 0)),
    ],
    out_shape=[
        jax.ShapeDtypeStruct((_M, _D), jnp.float32),
        jax.ShapeDtypeStruct((_M, _D), jnp.float32),
    ],
)


def _gather2_body(index_hbm, t1_hbm, t2_hbm, o1, o2, idx_v, bufs, *sems):
    sem_g = sems[:_NBUF]
    sem_w = sems[_NBUF:]
    wid = lax.axis_index("s") * _NC + lax.axis_index("c")
    base = wid * _BPW
    for j in range(_NCH):
        pltpu.sync_copy(index_hbm.at[pl.ds(base + j * _CHUNK, _CHUNK)],
                        idx_v.at[j])

    tasks = [(t1_hbm, o1, 0), (t2_hbm, o2, 0),
             (t1_hbm, o1, 1), (t2_hbm, o2, 1)]
    gathers = [None] * len(tasks)
    writes = [None] * len(tasks)

    def fire_gather(t):
        tbl, _, h = tasks[t]
        b = t % _NBUF
        cps = []
        for k in range(2):
            cps.append(pltpu.async_copy(
                tbl.at[idx_v.at[2 * h + k]],
                bufs.at[b].at[pl.ds(k * _CHUNK, _CHUNK)],
                sem_g[b]))
        gathers[t] = cps

    def fire_write(t):
        _, out, h = tasks[t]
        b = t % _NBUF
        for cp in gathers[t]:
            cp.wait()
        writes[t] = pltpu.async_copy(
            bufs.at[b], out.at[pl.ds(base + h * _HALF, _HALF)], sem_w[b])

    for t in range(len(tasks)):
        if t >= _NBUF:
            writes[t - _NBUF].wait()
        fire_gather(t)
        if t >= 1:
            fire_write(t - 1)
    fire_write(len(tasks) - 1)
    for t in range(max(0, len(tasks) - _NBUF), len(tasks)):
        writes[t].wait()


def _make_gather2():
    return functools.partial(
        pl.kernel,
        mesh=plsc.VectorSubcoreMesh(core_axis_name="c", subcore_axis_name="s"),
        out_type=[
            jax.ShapeDtypeStruct((_B, _D), jnp.float32),
            jax.ShapeDtypeStruct((_B, _D), jnp.float32),
        ],
        scratch_types=[
            pltpu.VMEM((_NCH, _CHUNK), jnp.int32),
            pltpu.VMEM((_NBUF, _HALF, _D), jnp.float32),
            pltpu.SemaphoreType.DMA,
            pltpu.SemaphoreType.DMA,
            pltpu.SemaphoreType.DMA,
            pltpu.SemaphoreType.DMA,
            pltpu.SemaphoreType.DMA,
            pltpu.SemaphoreType.DMA,
        ],
    )(_gather2_body)


_gather_feat = _make_gather2()
_gather_log = _make_gather2()


def kernel(x, index, weak_logits_mem, weak_features_mem,
           strong_logits_mem, strong_features_mem):
    del x
    wf, sf = _gather_feat(index, weak_features_mem, strong_features_mem)
    wl_p, sl_p = _pad2(weak_logits_mem, strong_logits_mem)
    wl, sl = _gather_log(index, wl_p, sl_p)
    return ([wl[:, :_C], sl[:, :_C]], [wf, sf])


# speedup vs baseline: 2.3178x; 1.0080x over previous
import functools

import jax
import jax.numpy as jnp
from jax import lax
from jax.experimental import pallas as pl
from jax.experimental.pallas import tpu as pltpu
from jax.experimental.pallas import tpu_sc as plsc

_M = 100000
_C = 100
_D = 128
_B = 16384

_info = plsc.get_sparse_core_info()
_NC, _NS = _info.num_cores, _info.num_subcores
_NW = _NC * _NS
_BPW = _B // _NW
_CHUNK = 128
_NCH = _BPW // _CHUNK
_HALF = _BPW // 2
_NBUF = 3

_PAD_ROWS = 10000


def _pad_body(wl_ref, sl_ref, wlp_ref, slp_ref):
    cfg = ((0, 0, 0), (0, _D - _C, 0))
    wlp_ref[...] = lax.pad(wl_ref[...], jnp.float32(0), cfg)
    slp_ref[...] = lax.pad(sl_ref[...], jnp.float32(0), cfg)


_pad2 = pl.pallas_call(
    _pad_body,
    grid=(_M // _PAD_ROWS,),
    in_specs=[
        pl.BlockSpec((_PAD_ROWS, _C), lambda i: (i, 0)),
        pl.BlockSpec((_PAD_ROWS, _C), lambda i: (i, 0)),
    ],
    out_specs=[
        pl.BlockSpec((_PAD_ROWS, _D), lambda i: (i, 0)),
        pl.BlockSpec((_PAD_ROWS, _D), lambda i: (i, 0)),
    ],
    out_shape=[
        jax.ShapeDtypeStruct((_M, _D), jnp.float32),
        jax.ShapeDtypeStruct((_M, _D), jnp.float32),
    ],
)


def _gather2_body(index_hbm, t1_hbm, t2_hbm, o1, o2, idx_v, bufs, *sems):
    sem_g = sems[:_NBUF]
    sem_w = sems[_NBUF:]
    wid = lax.axis_index("s") * _NC + lax.axis_index("c")
    base = wid * _BPW
    for j in range(_NCH):
        pltpu.sync_copy(index_hbm.at[pl.ds(base + j * _CHUNK, _CHUNK)],
                        idx_v.at[j])

    tasks = [(t1_hbm, o1, 0), (t2_hbm, o2, 0),
             (t1_hbm, o1, 1), (t2_hbm, o2, 1)]
    gathers = [None] * len(tasks)
    writes = [None] * len(tasks)

    def fire_gather(t):
        tbl, _, h = tasks[t]
        b = t % _NBUF
        cps = []
        for k in range(2):
            cps.append(pltpu.async_copy(
                tbl.at[idx_v.at[2 * h + k]],
                bufs.at[b].at[pl.ds(k * _CHUNK, _CHUNK)],
                sem_g[b]))
        gathers[t] = cps

    def fire_write(t):
        _, out, h = tasks[t]
        b = t % _NBUF
        for cp in gathers[t]:
            cp.wait()
        writes[t] = pltpu.async_copy(
            bufs.at[b], out.at[pl.ds(base + h * _HALF, _HALF)], sem_w[b])

    for t in range(len(tasks)):
        if t >= _NBUF:
            writes[t - _NBUF].wait()
        fire_gather(t)
        if t >= 1:
            fire_write(t - 1)
    fire_write(len(tasks) - 1)
    for t in range(max(0, len(tasks) - _NBUF), len(tasks)):
        writes[t].wait()


def _make_gather2():
    return functools.partial(
        pl.kernel,
        mesh=plsc.VectorSubcoreMesh(core_axis_name="c", subcore_axis_name="s"),
        out_type=[
            jax.ShapeDtypeStruct((_B, _D), jnp.float32),
            jax.ShapeDtypeStruct((_B, _D), jnp.float32),
        ],
        scratch_types=[
            pltpu.VMEM((_NCH, _CHUNK), jnp.int32),
            pltpu.VMEM((_NBUF, _HALF, _D), jnp.float32),
            pltpu.SemaphoreType.DMA,
            pltpu.SemaphoreType.DMA,
            pltpu.SemaphoreType.DMA,
            pltpu.SemaphoreType.DMA,
            pltpu.SemaphoreType.DMA,
            pltpu.SemaphoreType.DMA,
        ],
    )(_gather2_body)


_gather_feat = _make_gather2()
_gather_log = _make_gather2()


def kernel(x, index, weak_logits_mem, weak_features_mem,
           strong_logits_mem, strong_features_mem):
    del x
    wf, sf = _gather_feat(index, weak_features_mem, strong_features_mem)
    wl_p, sl_p = _pad2(weak_logits_mem, strong_logits_mem)
    wl, sl = _gather_log(index, wl_p, sl_p)
    return ([wl[:, :_C], sl[:, :_C]], [wf, sf])
